# baseline (speedup 1.0000x reference)
import functools

import jax
import jax.numpy as jnp
from jax import lax
from jax.experimental import pallas as pl
from jax.experimental.pallas import tpu as pltpu
from jax.experimental.pallas import tpu_sc as plsc

_WINDOW = 200
_NUM_CORES = 2
_NUM_SUBCORES = 16
_NW = _NUM_CORES * _NUM_SUBCORES
_LANES = 16
_SHIFTS = 8


@functools.partial(jax.jit, static_argnums=(2, 3, 4))
def _sc_window_gather(tables, start, batch, channels, b_per_w):
    rows_ext = tables.shape[1]
    mesh = plsc.VectorSubcoreMesh(
        core_axis_name="c",
        subcore_axis_name="s",
        num_cores=_NUM_CORES,
        num_subcores=_NUM_SUBCORES,
    )

    @functools.partial(
        pl.kernel,
        mesh=mesh,
        out_type=jax.ShapeDtypeStruct((batch, _WINDOW, channels), jnp.float32),
        scratch_types=[
            pltpu.VMEM((b_per_w,), jnp.int32),
            pltpu.VMEM_SHARED((_SHIFTS, rows_ext, channels), jnp.float32),
            pltpu.SemaphoreType.DMA,
            pltpu.SemaphoreType.DMA,
        ],
        compiler_params=pltpu.CompilerParams(use_tc_tiling_on_sc=True),
    )
    def k(tbl_hbm, start_hbm, out_hbm, idx_v, tbl_sp, sem_idx, sem_out):
        sid = lax.axis_index("s")
        wid = sid * _NUM_CORES + lax.axis_index("c")
        base = wid * b_per_w
        idx_cp = pltpu.make_async_copy(
            start_hbm.at[pl.ds(base, b_per_w)], idx_v, sem_idx
        )
        idx_cp.start()

        @pl.when(sid == 0)
        def _():
            pltpu.make_async_copy(tbl_hbm, tbl_sp, sem_out).start()
            pltpu.make_async_copy(tbl_hbm, tbl_sp, sem_out).wait()

        idx_cp.wait()
        plsc.subcore_barrier()

        def fire(g, carry):
            vec = idx_v[pl.ds(g * _LANES, _LANES)]
            shift = lax.rem(vec, _SHIFTS)
            aligned = vec - shift
            for lane in range(_LANES):
                pltpu.make_async_copy(
                    tbl_sp.at[
                        shift[lane],
                        pl.ds(pl.multiple_of(aligned[lane], _SHIFTS), _WINDOW),
                    ],
                    out_hbm.at[base + g * _LANES + lane],
                    sem_out,
                ).start()
            return carry

        lax.fori_loop(0, b_per_w // _LANES, fire, 0)

        def drain(b, carry):
            pltpu.make_async_copy(
                tbl_sp.at[0, pl.ds(0, _WINDOW)], out_hbm.at[base + b], sem_out
            ).wait()
            return carry

        lax.fori_loop(0, b_per_w, drain, 0)

    return k(tables, start)


def kernel(index, length, data):
    cycle_len, channels = data.shape
    batch = index.shape[0]
    start = jnp.asarray(
        (index.astype(jnp.int32) + (length - _WINDOW)) % cycle_len, jnp.int32
    )
    rows_ext = cycle_len + _WINDOW
    data_ext = jnp.concatenate([data, data[: _WINDOW + _SHIFTS]], axis=0)
    tables = jnp.stack([data_ext[k : k + rows_ext] for k in range(_SHIFTS)])
    return _sc_window_gather(tables, start, batch, channels, batch // _NW)
